# baseline (speedup 1.0000x reference)
import functools

import jax
import jax.numpy as jnp
from jax import lax
from jax.experimental import pallas as pl
from jax.experimental.pallas import tpu as pltpu

_N = 10000
_HEADS1 = 4
_HID = 256
_OUT = 256


def _mm2_body(x_ref, wl_ref, bl_ref, wr_ref, br_ref, xl_ref, xr_ref):
    x = x_ref[...]
    xl_ref[...] = (
        jnp.dot(x, wl_ref[...], preferred_element_type=jnp.float32) + bl_ref[...]
    )
    xr_ref[...] = (
        jnp.dot(x, wr_ref[...], preferred_element_type=jnp.float32) + br_ref[...]
    )


def _proj2(x, Wl, bl, Wr, br, elu_in=False, bias_pre=None):
    n, k = x.shape
    ko = Wl.shape[1]
    blk = 1000
    grid = n // blk

    if elu_in:
        def body(x_ref, bp_ref, wl_ref, bl_ref, wr_ref, br_ref, xl_ref, xr_ref):
            xv = x_ref[...] + bp_ref[...]
            xv = jnp.where(xv > 0, xv, jnp.exp(jnp.minimum(xv, 0.0)) - 1.0)
            xl_ref[...] = (
                jnp.dot(xv, wl_ref[...], preferred_element_type=jnp.float32)
                + bl_ref[...]
            )
            xr_ref[...] = (
                jnp.dot(xv, wr_ref[...], preferred_element_type=jnp.float32)
                + br_ref[...]
            )

        in_specs = [
            pl.BlockSpec((blk, k), lambda i: (i, 0)),
            pl.BlockSpec((1, k), lambda i: (0, 0)),
            pl.BlockSpec((k, ko), lambda i: (0, 0)),
            pl.BlockSpec((1, ko), lambda i: (0, 0)),
            pl.BlockSpec((k, ko), lambda i: (0, 0)),
            pl.BlockSpec((1, ko), lambda i: (0, 0)),
        ]
        args = (x, bias_pre.reshape(1, k), Wl, bl.reshape(1, ko), Wr, br.reshape(1, ko))
    else:
        body = _mm2_body
        in_specs = [
            pl.BlockSpec((blk, k), lambda i: (i, 0)),
            pl.BlockSpec((k, ko), lambda i: (0, 0)),
            pl.BlockSpec((1, ko), lambda i: (0, 0)),
            pl.BlockSpec((k, ko), lambda i: (0, 0)),
            pl.BlockSpec((1, ko), lambda i: (0, 0)),
        ]
        args = (x, Wl, bl.reshape(1, ko), Wr, br.reshape(1, ko))

    xl, xr = pl.pallas_call(
        body,
        grid=(grid,),
        in_specs=in_specs,
        out_specs=[
            pl.BlockSpec((blk, ko), lambda i: (i, 0)),
            pl.BlockSpec((blk, ko), lambda i: (i, 0)),
        ],
        out_shape=[
            jax.ShapeDtypeStruct((n, ko), jnp.float32),
            jax.ShapeDtypeStruct((n, ko), jnp.float32),
        ],
    )(*args)
    return xl, xr


def _edge_phase_xla(xl, xr, src, dst, att_flat, heads, outc):
    n = xl.shape[0]
    xl3 = xl.reshape(n, heads, outc)
    xr3 = xr.reshape(n, heads, outc)
    att = att_flat.reshape(heads, outc)
    m = jax.nn.leaky_relu(xl3[src] + xr3[dst], negative_slope=0.2)
    e = jnp.sum(m * att[None, :, :], axis=-1)
    emax = jax.ops.segment_max(e, dst, num_segments=n)
    emax = jnp.where(jnp.isfinite(emax), emax, 0.0)
    ee = jnp.exp(e - emax[dst])
    denom = jax.ops.segment_sum(ee, dst, num_segments=n)
    alpha = ee / (denom[dst] + 1e-16)
    out = jax.ops.segment_sum(xl3[src] * alpha[:, :, None], dst, num_segments=n)
    return out.reshape(n, heads * outc)


def kernel(x, edge_index, W1l, b1l, W1r, b1r, att1, bias1, W2l, b2l, W2r, b2r, att2, bias2):
    n = x.shape[0]
    ar = jnp.arange(n, dtype=edge_index.dtype)
    ei = jnp.concatenate([edge_index, jnp.stack([ar, ar])], axis=1)
    src, dst = ei[0], ei[1]

    xl1, xr1 = _proj2(x, W1l, b1l, W1r, b1r)
    agg1 = _edge_phase_xla(xl1, xr1, src, dst, att1.reshape(-1), _HEADS1, _HID)

    xl2, xr2 = _proj2(agg1, W2l, b2l, W2r, b2r, elu_in=True, bias_pre=bias1)
    agg2 = _edge_phase_xla(xl2, xr2, src, dst, att2.reshape(-1), 1, _OUT)

    return agg2 + bias2


# speedup vs baseline: 4.9184x; 4.9184x over previous
import functools

import jax
import jax.numpy as jnp
from jax import lax
from jax.experimental import pallas as pl
from jax.experimental.pallas import tpu as pltpu
from jax.experimental.pallas import tpu_sc as plsc

_N = 10000
_HEADS1 = 4
_HID = 256
_OUT = 256

_NC = 2
_NS = 16
_NW = _NC * _NS
_NPT = -(-_N // _NW)
_ECAP = 8192
_ECH = 16


def _proj2(x, Wl, bl, Wr, br, elu_in=False, bias_pre=None):
    n, k = x.shape
    ko = Wl.shape[1]
    blk = 1000
    grid = n // blk

    if elu_in:
        def body(x_ref, bp_ref, wl_ref, bl_ref, wr_ref, br_ref, xl_ref, xr_ref):
            xv = x_ref[...] + bp_ref[...]
            xv = jnp.where(xv > 0, xv, jnp.exp(jnp.minimum(xv, 0.0)) - 1.0)
            xl_ref[...] = (
                jnp.dot(xv, wl_ref[...], preferred_element_type=jnp.float32)
                + bl_ref[...]
            )
            xr_ref[...] = (
                jnp.dot(xv, wr_ref[...], preferred_element_type=jnp.float32)
                + br_ref[...]
            )

        in_specs = [
            pl.BlockSpec((blk, k), lambda i: (i, 0)),
            pl.BlockSpec((1, k), lambda i: (0, 0)),
            pl.BlockSpec((k, ko), lambda i: (0, 0)),
            pl.BlockSpec((1, ko), lambda i: (0, 0)),
            pl.BlockSpec((k, ko), lambda i: (0, 0)),
            pl.BlockSpec((1, ko), lambda i: (0, 0)),
        ]
        args = (x, bias_pre.reshape(1, k), Wl, bl.reshape(1, ko), Wr, br.reshape(1, ko))
    else:
        def body(x_ref, wl_ref, bl_ref, wr_ref, br_ref, xl_ref, xr_ref):
            xv = x_ref[...]
            xl_ref[...] = (
                jnp.dot(xv, wl_ref[...], preferred_element_type=jnp.float32)
                + bl_ref[...]
            )
            xr_ref[...] = (
                jnp.dot(xv, wr_ref[...], preferred_element_type=jnp.float32)
                + br_ref[...]
            )

        in_specs = [
            pl.BlockSpec((blk, k), lambda i: (i, 0)),
            pl.BlockSpec((k, ko), lambda i: (0, 0)),
            pl.BlockSpec((1, ko), lambda i: (0, 0)),
            pl.BlockSpec((k, ko), lambda i: (0, 0)),
            pl.BlockSpec((1, ko), lambda i: (0, 0)),
        ]
        args = (x, Wl, bl.reshape(1, ko), Wr, br.reshape(1, ko))

    return pl.pallas_call(
        body,
        grid=(grid,),
        in_specs=in_specs,
        out_specs=[
            pl.BlockSpec((blk, ko), lambda i: (i, 0)),
            pl.BlockSpec((blk, ko), lambda i: (i, 0)),
        ],
        out_shape=[
            jax.ShapeDtypeStruct((n, ko), jnp.float32),
            jax.ShapeDtypeStruct((n, ko), jnp.float32),
        ],
    )(*args)


def _shufsum(v):
    for sh in (8, 4, 2, 1):
        perm = (lax.iota(jnp.int32, 16) + sh) & 15
        v = v + jnp.take(v, perm)
    return v


def _ilane(vec, k):
    sel = jnp.where(lax.iota(jnp.int32, 16) == k, vec, 0)
    return _shufsum(sel)[0]


def _make_sc_edge(heads, outc, n):
    hc = heads * outc
    cpsl = outc // 16

    def body(xl_hbm, xr_hbm, src_hbm, dst_hbm, meta_hbm, att_hbm, out_hbm,
             srcb, dstb, attb, metab, xlb, xrb, ob,
             sxl0, sxl1, sxr0, sxr1):
        sems = ((sxl0, sxr0), (sxl1, sxr1))
        wid = lax.axis_index("c") * _NS + lax.axis_index("s")

        pltpu.sync_copy(meta_hbm.at[wid], metab)
        mv = metab[...]
        e_lo = mv[0]
        e_hi = mv[1]
        base = pl.multiple_of(e_lo & ~7, 8)
        total = e_hi - base
        nch = (total + _ECH - 1) // _ECH

        pltpu.sync_copy(src_hbm.at[pl.ds(base, _ECAP)], srcb)
        pltpu.sync_copy(dst_hbm.at[pl.ds(base, _ECAP)], dstb)
        pltpu.sync_copy(att_hbm, attb)

        zeros16 = jnp.zeros((16,), jnp.float32)

        def zero_ob():
            def zb(s, c):
                ob[pl.ds(s * 16, 16)] = zeros16
                return c
            lax.fori_loop(0, hc // 16, zb, 0)

        zero_ob()

        def issue(c, b):
            @pl.when(c < nch)
            def _():
                idx = srcb[pl.ds(_ECH * c, _ECH)]
                pltpu.async_copy(xl_hbm.at[idx], xlb.at[b], sems[b][0])
                idxd = dstb[pl.ds(_ECH * c, _ECH)]
                pltpu.async_copy(xr_hbm.at[idxd], xrb.at[b], sems[b][1])

        def wait(c, b):
            @pl.when(c < nch)
            def _():
                idx = srcb[pl.ds(_ECH * c, _ECH)]
                pltpu.make_async_copy(xl_hbm.at[idx], xlb.at[b], sems[b][0]).wait()
                idxd = dstb[pl.ds(_ECH * c, _ECH)]
                pltpu.make_async_copy(xr_hbm.at[idxd], xrb.at[b], sems[b][1]).wait()

        def finalize(cur, d):
            for h in range(heads):
                inv_v = 1.0 / (d[h] + 1e-16)

                def nb(s, c):
                    off = h * outc + s * 16
                    ob[pl.ds(off, 16)] = ob[pl.ds(off, 16)] * inv_v
                    return c
                lax.fori_loop(0, cpsl, nb, 0)
            pltpu.sync_copy(ob, out_hbm.at[cur])
            zero_ob()

        neg_big_v = jnp.full((16,), -1e30, jnp.float32)

        def process_chunk(c, b, carry):
            wait(c, b)
            dst16 = dstb[pl.ds(_ECH * c, _ECH)]
            ebase = base + _ECH * c
            k_lo = jnp.maximum(0, e_lo - ebase)
            k_hi = jnp.minimum(_ECH, e_hi - ebase)

            def edge_body(k, carry):
                cur, m, d = carry
                dstv = _ilane(dst16, k)
                is_new = dstv != cur

                @pl.when(jnp.logical_and(is_new, cur >= 0))
                def _():
                    finalize(cur, d)

                m = tuple(jnp.where(is_new, neg_big_v, m[h]) for h in range(heads))
                d = tuple(jnp.where(is_new, zeros16, d[h]) for h in range(heads))
                cur = dstv

                m_out = []
                d_out = []
                for h in range(heads):
                    def eb(s, acc):
                        off = h * outc + s * 16
                        zl = xlb[b, k, pl.ds(off, 16)]
                        zr = xrb[b, k, pl.ds(off, 16)]
                        z = zl + zr
                        z = jnp.maximum(z, 0.2 * z)
                        return acc + attb[pl.ds(off, 16)] * z

                    acc = lax.fori_loop(0, cpsl, eb, zeros16)
                    e_v = _shufsum(acc)
                    m_new = jnp.maximum(m[h], e_v)
                    scale_v = jnp.exp(m[h] - m_new)
                    w_v = jnp.exp(e_v - m_new)
                    d_new = d[h] * scale_v + w_v

                    def ub(s, cc):
                        off = h * outc + s * 16
                        ob[pl.ds(off, 16)] = (
                            ob[pl.ds(off, 16)] * scale_v
                            + w_v * xlb[b, k, pl.ds(off, 16)]
                        )
                        return cc
                    lax.fori_loop(0, cpsl, ub, 0)
                    m_out.append(m_new)
                    d_out.append(d_new)

                return cur, tuple(m_out), tuple(d_out)

            carry = lax.fori_loop(k_lo, k_hi, edge_body, carry)
            issue(c + 2, b)
            return carry

        issue(0, 0)
        issue(1, 1)

        carry0 = (
            jnp.int32(-1),
            tuple(neg_big_v for _ in range(heads)),
            tuple(zeros16 for _ in range(heads)),
        )

        def pair_body(g, carry):
            carry = process_chunk(2 * g, 0, carry)
            carry = process_chunk(2 * g + 1, 1, carry)
            return carry

        cur, m, d = lax.fori_loop(0, (nch + 1) // 2, pair_body, carry0)

        @pl.when(cur >= 0)
        def _():
            finalize(cur, d)

    mesh = plsc.VectorSubcoreMesh(
        core_axis_name="c", subcore_axis_name="s", num_cores=_NC, num_subcores=_NS
    )
    return pl.kernel(
        body,
        out_type=jax.ShapeDtypeStruct((n, hc), jnp.float32),
        mesh=mesh,
        scratch_types=[
            pltpu.VMEM((_ECAP,), jnp.int32),
            pltpu.VMEM((_ECAP,), jnp.int32),
            pltpu.VMEM((hc,), jnp.float32),
            pltpu.VMEM((16,), jnp.int32),
            pltpu.VMEM((2, _ECH, hc), jnp.float32),
            pltpu.VMEM((2, _ECH, hc), jnp.float32),
            pltpu.VMEM((hc,), jnp.float32),
            pltpu.SemaphoreType.DMA,
            pltpu.SemaphoreType.DMA,
            pltpu.SemaphoreType.DMA,
            pltpu.SemaphoreType.DMA,
        ],
    )


def _edge_phase_sc(xl, xr, src_s, dst_s, meta, att_flat, heads, outc):
    n = xl.shape[0]
    fn = _make_sc_edge(heads, outc, n)
    return fn(xl, xr, src_s, dst_s, meta, att_flat)


def _prep_edges(edge_index, n):
    ar = jnp.arange(n, dtype=edge_index.dtype)
    ei = jnp.concatenate([edge_index, jnp.stack([ar, ar])], axis=1)
    src, dst = ei[0], ei[1]
    order = jnp.argsort(dst)
    src_s = src[order]
    dst_s = dst[order]
    e = src_s.shape[0]
    pad = _ECAP + 16
    src_s = jnp.concatenate([src_s, jnp.zeros((pad,), src_s.dtype)])
    dst_s = jnp.concatenate([dst_s, jnp.full((pad,), n - 1, dst_s.dtype)])
    bounds = jnp.minimum(jnp.arange(_NW + 1, dtype=jnp.int32) * _NPT, n)
    offs = jnp.searchsorted(dst_s[:e], bounds).astype(jnp.int32)
    meta = jnp.zeros((_NW, 16), jnp.int32)
    meta = meta.at[:, 0].set(offs[:-1])
    meta = meta.at[:, 1].set(offs[1:])
    return src_s, dst_s, meta


def kernel(x, edge_index, W1l, b1l, W1r, b1r, att1, bias1, W2l, b2l, W2r, b2r, att2, bias2):
    n = x.shape[0]
    src_s, dst_s, meta = _prep_edges(edge_index, n)

    xl1, xr1 = _proj2(x, W1l, b1l, W1r, b1r)
    agg1 = _edge_phase_sc(xl1, xr1, src_s, dst_s, meta, att1.reshape(-1), _HEADS1, _HID)

    xl2, xr2 = _proj2(agg1, W2l, b2l, W2r, b2r, elu_in=True, bias_pre=bias1)
    agg2 = _edge_phase_sc(xl2, xr2, src_s, dst_s, meta, att2.reshape(-1), 1, _OUT)

    return agg2 + bias2


# speedup vs baseline: 6.4096x; 1.3032x over previous
import functools

import jax
import jax.numpy as jnp
from jax import lax
from jax.experimental import pallas as pl
from jax.experimental.pallas import tpu as pltpu
from jax.experimental.pallas import tpu_sc as plsc

_N = 10000
_HEADS1 = 4
_HID = 256
_OUT = 256

_NC = 2
_NS = 16
_NW = _NC * _NS
_NPT = -(-_N // _NW)
_ECAP = 8192
_ECH = 16


def _proj2(x, Wl, bl, Wr, br, elu_in=False, bias_pre=None):
    n, k = x.shape
    ko = Wl.shape[1]
    blk = 1000
    grid = n // blk

    if elu_in:
        def body(x_ref, bp_ref, wl_ref, bl_ref, wr_ref, br_ref, xl_ref, xr_ref):
            xv = x_ref[...] + bp_ref[...]
            xv = jnp.where(xv > 0, xv, jnp.exp(jnp.minimum(xv, 0.0)) - 1.0)
            xl_ref[...] = (
                jnp.dot(xv, wl_ref[...], preferred_element_type=jnp.float32)
                + bl_ref[...]
            )
            xr_ref[...] = (
                jnp.dot(xv, wr_ref[...], preferred_element_type=jnp.float32)
                + br_ref[...]
            )

        in_specs = [
            pl.BlockSpec((blk, k), lambda i: (i, 0)),
            pl.BlockSpec((1, k), lambda i: (0, 0)),
            pl.BlockSpec((k, ko), lambda i: (0, 0)),
            pl.BlockSpec((1, ko), lambda i: (0, 0)),
            pl.BlockSpec((k, ko), lambda i: (0, 0)),
            pl.BlockSpec((1, ko), lambda i: (0, 0)),
        ]
        args = (x, bias_pre.reshape(1, k), Wl, bl.reshape(1, ko), Wr, br.reshape(1, ko))
    else:
        def body(x_ref, wl_ref, bl_ref, wr_ref, br_ref, xl_ref, xr_ref):
            xv = x_ref[...]
            xl_ref[...] = (
                jnp.dot(xv, wl_ref[...], preferred_element_type=jnp.float32)
                + bl_ref[...]
            )
            xr_ref[...] = (
                jnp.dot(xv, wr_ref[...], preferred_element_type=jnp.float32)
                + br_ref[...]
            )

        in_specs = [
            pl.BlockSpec((blk, k), lambda i: (i, 0)),
            pl.BlockSpec((k, ko), lambda i: (0, 0)),
            pl.BlockSpec((1, ko), lambda i: (0, 0)),
            pl.BlockSpec((k, ko), lambda i: (0, 0)),
            pl.BlockSpec((1, ko), lambda i: (0, 0)),
        ]
        args = (x, Wl, bl.reshape(1, ko), Wr, br.reshape(1, ko))

    return pl.pallas_call(
        body,
        grid=(grid,),
        in_specs=in_specs,
        out_specs=[
            pl.BlockSpec((blk, ko), lambda i: (i, 0)),
            pl.BlockSpec((blk, ko), lambda i: (i, 0)),
        ],
        out_shape=[
            jax.ShapeDtypeStruct((n, ko), jnp.float32),
            jax.ShapeDtypeStruct((n, ko), jnp.float32),
        ],
    )(*args)


def _shufsum(v):
    for sh in (8, 4, 2, 1):
        perm = (lax.iota(jnp.int32, 16) + sh) & 15
        v = v + jnp.take(v, perm)
    return v


def _ilane(vec, k):
    sel = jnp.where(lax.iota(jnp.int32, 16) == k, vec, 0)
    return _shufsum(sel)[0]


def _make_sc_edge(heads, outc, n):
    hc = heads * outc
    cpsl = outc // 16

    def body(xl_hbm, xr_hbm, src_hbm, dst_hbm, meta_hbm, att_hbm, out_hbm,
             srcb, dstb, attb, metab, xlb, xrb, ob,
             sxl0, sxl1, sxr0, sxr1):
        sems = ((sxl0, sxr0), (sxl1, sxr1))
        wid = lax.axis_index("c") * _NS + lax.axis_index("s")

        pltpu.sync_copy(meta_hbm.at[wid], metab)
        mv = metab[...]
        e_lo = mv[0]
        e_hi = mv[1]
        base = pl.multiple_of(e_lo & ~7, 8)
        total = e_hi - base
        nch = (total + _ECH - 1) // _ECH

        pltpu.sync_copy(src_hbm.at[pl.ds(base, _ECAP)], srcb)
        pltpu.sync_copy(dst_hbm.at[pl.ds(base, _ECAP)], dstb)
        pltpu.sync_copy(att_hbm, attb)

        zeros16 = jnp.zeros((16,), jnp.float32)

        def zero_ob():
            def zb(s, c):
                ob[pl.ds(s * 16, 16)] = zeros16
                return c
            lax.fori_loop(0, hc // 16, zb, 0)

        zero_ob()

        def issue(c, b):
            @pl.when(c < nch)
            def _():
                idx = srcb[pl.ds(_ECH * c, _ECH)]
                pltpu.async_copy(xl_hbm.at[idx], xlb.at[b], sems[b][0])
                idxd = dstb[pl.ds(_ECH * c, _ECH)]
                pltpu.async_copy(xr_hbm.at[idxd], xrb.at[b], sems[b][1])

        def wait(c, b):
            @pl.when(c < nch)
            def _():
                idx = srcb[pl.ds(_ECH * c, _ECH)]
                pltpu.make_async_copy(xl_hbm.at[idx], xlb.at[b], sems[b][0]).wait()
                idxd = dstb[pl.ds(_ECH * c, _ECH)]
                pltpu.make_async_copy(xr_hbm.at[idxd], xrb.at[b], sems[b][1]).wait()

        def finalize(cur, d):
            for h in range(heads):
                inv_v = 1.0 / (d[h] + 1e-16)

                def nb(s4, c):
                    for j in range(4):
                        off = h * outc + (s4 * 4 + j) * 16
                        ob[pl.ds(off, 16)] = ob[pl.ds(off, 16)] * inv_v
                    return c
                lax.fori_loop(0, cpsl // 4, nb, 0)
            pltpu.sync_copy(ob, out_hbm.at[cur])

        neg_big_v = jnp.full((16,), -1e30, jnp.float32)

        def process_chunk(c, b, carry):
            wait(c, b)
            dst16 = dstb[pl.ds(_ECH * c, _ECH)]
            ebase = base + _ECH * c
            k_lo = jnp.maximum(0, e_lo - ebase)
            k_hi = jnp.minimum(_ECH, e_hi - ebase)

            def edge_body(k, carry):
                cur, m, d = carry
                dstv = _ilane(dst16, k)
                is_new = dstv != cur

                @pl.when(jnp.logical_and(is_new, cur >= 0))
                def _():
                    finalize(cur, d)

                m = tuple(jnp.where(is_new, neg_big_v, m[h]) for h in range(heads))
                d = tuple(jnp.where(is_new, zeros16, d[h]) for h in range(heads))
                cur = dstv

                m_out = []
                d_out = []
                for h in range(heads):
                    def eb(s4, acc):
                        for j in range(4):
                            off = h * outc + (s4 * 4 + j) * 16
                            zl = xlb[b, k, pl.ds(off, 16)]
                            zr = xrb[b, k, pl.ds(off, 16)]
                            z = zl + zr
                            z = jnp.maximum(z, 0.2 * z)
                            acc = acc + attb[pl.ds(off, 16)] * z
                        return acc

                    acc = lax.fori_loop(0, cpsl // 4, eb, zeros16)
                    e_v = _shufsum(acc)
                    m_new = jnp.maximum(m[h], e_v)
                    scale_v = jnp.exp(m[h] - m_new)
                    w_v = jnp.exp(e_v - m_new)
                    d_new = d[h] * scale_v + w_v

                    def ub(s4, cc):
                        for j in range(4):
                            off = h * outc + (s4 * 4 + j) * 16
                            ob[pl.ds(off, 16)] = (
                                ob[pl.ds(off, 16)] * scale_v
                                + w_v * xlb[b, k, pl.ds(off, 16)]
                            )
                        return cc
                    lax.fori_loop(0, cpsl // 4, ub, 0)
                    m_out.append(m_new)
                    d_out.append(d_new)

                return cur, tuple(m_out), tuple(d_out)

            carry = lax.fori_loop(k_lo, k_hi, edge_body, carry)
            issue(c + 2, b)
            return carry

        issue(0, 0)
        issue(1, 1)

        carry0 = (
            jnp.int32(-1),
            tuple(neg_big_v for _ in range(heads)),
            tuple(zeros16 for _ in range(heads)),
        )

        def pair_body(g, carry):
            carry = process_chunk(2 * g, 0, carry)
            carry = process_chunk(2 * g + 1, 1, carry)
            return carry

        cur, m, d = lax.fori_loop(0, (nch + 1) // 2, pair_body, carry0)

        @pl.when(cur >= 0)
        def _():
            finalize(cur, d)

    mesh = plsc.VectorSubcoreMesh(
        core_axis_name="c", subcore_axis_name="s", num_cores=_NC, num_subcores=_NS
    )
    return pl.kernel(
        body,
        out_type=jax.ShapeDtypeStruct((n, hc), jnp.float32),
        mesh=mesh,
        scratch_types=[
            pltpu.VMEM((_ECAP,), jnp.int32),
            pltpu.VMEM((_ECAP,), jnp.int32),
            pltpu.VMEM((hc,), jnp.float32),
            pltpu.VMEM((16,), jnp.int32),
            pltpu.VMEM((2, _ECH, hc), jnp.float32),
            pltpu.VMEM((2, _ECH, hc), jnp.float32),
            pltpu.VMEM((hc,), jnp.float32),
            pltpu.SemaphoreType.DMA,
            pltpu.SemaphoreType.DMA,
            pltpu.SemaphoreType.DMA,
            pltpu.SemaphoreType.DMA,
        ],
    )


def _edge_phase_sc(xl, xr, src_s, dst_s, meta, att_flat, heads, outc):
    n = xl.shape[0]
    fn = _make_sc_edge(heads, outc, n)
    return fn(xl, xr, src_s, dst_s, meta, att_flat)


def _prep_edges(edge_index, n):
    ar = jnp.arange(n, dtype=edge_index.dtype)
    ei = jnp.concatenate([edge_index, jnp.stack([ar, ar])], axis=1)
    src, dst = ei[0], ei[1]
    order = jnp.argsort(dst)
    src_s = src[order]
    dst_s = dst[order]
    e = src_s.shape[0]
    pad = _ECAP + 16
    src_s = jnp.concatenate([src_s, jnp.zeros((pad,), src_s.dtype)])
    dst_s = jnp.concatenate([dst_s, jnp.full((pad,), n - 1, dst_s.dtype)])
    bounds = jnp.minimum(jnp.arange(_NW + 1, dtype=jnp.int32) * _NPT, n)
    offs = jnp.searchsorted(dst_s[:e], bounds).astype(jnp.int32)
    meta = jnp.zeros((_NW, 16), jnp.int32)
    meta = meta.at[:, 0].set(offs[:-1])
    meta = meta.at[:, 1].set(offs[1:])
    return src_s, dst_s, meta


def kernel(x, edge_index, W1l, b1l, W1r, b1r, att1, bias1, W2l, b2l, W2r, b2r, att2, bias2):
    n = x.shape[0]
    src_s, dst_s, meta = _prep_edges(edge_index, n)

    xl1, xr1 = _proj2(x, W1l, b1l, W1r, b1r)
    agg1 = _edge_phase_sc(xl1, xr1, src_s, dst_s, meta, att1.reshape(-1), _HEADS1, _HID)

    xl2, xr2 = _proj2(agg1, W2l, b2l, W2r, b2r, elu_in=True, bias_pre=bias1)
    agg2 = _edge_phase_sc(xl2, xr2, src_s, dst_s, meta, att2.reshape(-1), 1, _OUT)

    return agg2 + bias2


# speedup vs baseline: 6.4129x; 1.0005x over previous
import functools

import jax
import jax.numpy as jnp
from jax import lax
from jax.experimental import pallas as pl
from jax.experimental.pallas import tpu as pltpu
from jax.experimental.pallas import tpu_sc as plsc

_N = 10000
_HEADS1 = 4
_HID = 256
_OUT = 256

_NC = 2
_NS = 16
_NW = _NC * _NS
_NPT = -(-_N // _NW)
_ECAP = 8192
_ECH = 16


def _proj2(x, Wl, bl, Wr, br, elu_in=False, bias_pre=None):
    n, k = x.shape
    ko = Wl.shape[1]
    blk = 1000
    grid = n // blk

    if elu_in:
        def body(x_ref, bp_ref, wl_ref, bl_ref, wr_ref, br_ref, xl_ref, xr_ref):
            xv = x_ref[...] + bp_ref[...]
            xv = jnp.where(xv > 0, xv, jnp.exp(jnp.minimum(xv, 0.0)) - 1.0)
            xl_ref[...] = (
                jnp.dot(xv, wl_ref[...], preferred_element_type=jnp.float32)
                + bl_ref[...]
            )
            xr_ref[...] = (
                jnp.dot(xv, wr_ref[...], preferred_element_type=jnp.float32)
                + br_ref[...]
            )

        in_specs = [
            pl.BlockSpec((blk, k), lambda i: (i, 0)),
            pl.BlockSpec((1, k), lambda i: (0, 0)),
            pl.BlockSpec((k, ko), lambda i: (0, 0)),
            pl.BlockSpec((1, ko), lambda i: (0, 0)),
            pl.BlockSpec((k, ko), lambda i: (0, 0)),
            pl.BlockSpec((1, ko), lambda i: (0, 0)),
        ]
        args = (x, bias_pre.reshape(1, k), Wl, bl.reshape(1, ko), Wr, br.reshape(1, ko))
    else:
        def body(x_ref, wl_ref, bl_ref, wr_ref, br_ref, xl_ref, xr_ref):
            xv = x_ref[...]
            xl_ref[...] = (
                jnp.dot(xv, wl_ref[...], preferred_element_type=jnp.float32)
                + bl_ref[...]
            )
            xr_ref[...] = (
                jnp.dot(xv, wr_ref[...], preferred_element_type=jnp.float32)
                + br_ref[...]
            )

        in_specs = [
            pl.BlockSpec((blk, k), lambda i: (i, 0)),
            pl.BlockSpec((k, ko), lambda i: (0, 0)),
            pl.BlockSpec((1, ko), lambda i: (0, 0)),
            pl.BlockSpec((k, ko), lambda i: (0, 0)),
            pl.BlockSpec((1, ko), lambda i: (0, 0)),
        ]
        args = (x, Wl, bl.reshape(1, ko), Wr, br.reshape(1, ko))

    return pl.pallas_call(
        body,
        grid=(grid,),
        in_specs=in_specs,
        out_specs=[
            pl.BlockSpec((blk, ko), lambda i: (i, 0)),
            pl.BlockSpec((blk, ko), lambda i: (i, 0)),
        ],
        out_shape=[
            jax.ShapeDtypeStruct((n, ko), jnp.float32),
            jax.ShapeDtypeStruct((n, ko), jnp.float32),
        ],
    )(*args)


def _shufsum(v):
    for sh in (8, 4, 2, 1):
        perm = (lax.iota(jnp.int32, 16) + sh) & 15
        v = v + jnp.take(v, perm)
    return v


def _ilane(vec, k):
    sel = jnp.where(lax.iota(jnp.int32, 16) == k, vec, 0)
    return _shufsum(sel)[0]


def _make_sc_edge(heads, outc, n):
    hc = heads * outc
    cpsl = outc // 16

    def body(xl_hbm, xr_hbm, src_hbm, dst_hbm, meta_hbm, att_hbm, out_hbm,
             srcb, dstb, attb, metab, xlb, xrb, ob,
             sxl0, sxl1, sxr0, sxr1):
        sems = ((sxl0, sxr0), (sxl1, sxr1))
        wid = lax.axis_index("c") * _NS + lax.axis_index("s")

        pltpu.sync_copy(meta_hbm.at[wid], metab)
        mv = metab[...]
        e_lo = mv[0]
        e_hi = mv[1]
        base = pl.multiple_of(e_lo & ~7, 8)
        total = e_hi - base
        nch = (total + _ECH - 1) // _ECH

        pltpu.sync_copy(src_hbm.at[pl.ds(base, _ECAP)], srcb)
        pltpu.sync_copy(dst_hbm.at[pl.ds(base, _ECAP)], dstb)
        pltpu.sync_copy(att_hbm, attb)

        zeros16 = jnp.zeros((16,), jnp.float32)

        def zero_ob():
            def zb(s, c):
                ob[pl.ds(s * 16, 16)] = zeros16
                return c
            lax.fori_loop(0, hc // 16, zb, 0)

        zero_ob()

        def issue(c, b):
            @pl.when(c < nch)
            def _():
                idx = srcb[pl.ds(_ECH * c, _ECH)]
                pltpu.async_copy(xl_hbm.at[idx], xlb.at[b], sems[b][0])
                idxd = dstb[pl.ds(_ECH * c, _ECH)]
                pltpu.async_copy(xr_hbm.at[idxd], xrb.at[b], sems[b][1])

        def wait(c, b):
            @pl.when(c < nch)
            def _():
                idx = srcb[pl.ds(_ECH * c, _ECH)]
                pltpu.make_async_copy(xl_hbm.at[idx], xlb.at[b], sems[b][0]).wait()
                idxd = dstb[pl.ds(_ECH * c, _ECH)]
                pltpu.make_async_copy(xr_hbm.at[idxd], xrb.at[b], sems[b][1]).wait()

        def finalize(cur, d):
            for h in range(heads):
                inv_v = 1.0 / (d[h] + 1e-16)

                def nb(s4, c):
                    for j in range(4):
                        off = h * outc + (s4 * 4 + j) * 16
                        ob[pl.ds(off, 16)] = ob[pl.ds(off, 16)] * inv_v
                    return c
                lax.fori_loop(0, cpsl // 4, nb, 0)
            pltpu.sync_copy(ob, out_hbm.at[cur])

        neg_big_v = jnp.full((16,), -1e30, jnp.float32)

        def process_chunk(c, b, carry):
            wait(c, b)
            dst16 = dstb[pl.ds(_ECH * c, _ECH)]
            ebase = base + _ECH * c
            k_lo = jnp.maximum(0, e_lo - ebase)
            k_hi = jnp.minimum(_ECH, e_hi - ebase)

            def edge_body(k, carry):
                cur, m, d = carry
                dstv = _ilane(dst16, k)
                is_new = dstv != cur

                @pl.when(jnp.logical_and(is_new, cur >= 0))
                def _():
                    finalize(cur, d)

                m = tuple(jnp.where(is_new, neg_big_v, m[h]) for h in range(heads))
                d = tuple(jnp.where(is_new, zeros16, d[h]) for h in range(heads))
                cur = dstv

                m_out = []
                d_out = []
                for h in range(heads):
                    acc = zeros16
                    for s in range(cpsl):
                        off = h * outc + s * 16
                        zl = xlb[b, k, pl.ds(off, 16)]
                        zr = xrb[b, k, pl.ds(off, 16)]
                        z = zl + zr
                        z = jnp.maximum(z, 0.2 * z)
                        acc = acc + attb[pl.ds(off, 16)] * z

                    e_v = _shufsum(acc)
                    m_new = jnp.maximum(m[h], e_v)
                    scale_v = jnp.exp(m[h] - m_new)
                    w_v = jnp.exp(e_v - m_new)
                    d_new = d[h] * scale_v + w_v

                    for s in range(cpsl):
                        off = h * outc + s * 16
                        ob[pl.ds(off, 16)] = (
                            ob[pl.ds(off, 16)] * scale_v
                            + w_v * xlb[b, k, pl.ds(off, 16)]
                        )
                    m_out.append(m_new)
                    d_out.append(d_new)

                return cur, tuple(m_out), tuple(d_out)

            carry = lax.fori_loop(k_lo, k_hi, edge_body, carry)
            issue(c + 2, b)
            return carry

        issue(0, 0)
        issue(1, 1)

        carry0 = (
            jnp.int32(-1),
            tuple(neg_big_v for _ in range(heads)),
            tuple(zeros16 for _ in range(heads)),
        )

        def pair_body(g, carry):
            carry = process_chunk(2 * g, 0, carry)
            carry = process_chunk(2 * g + 1, 1, carry)
            return carry

        cur, m, d = lax.fori_loop(0, (nch + 1) // 2, pair_body, carry0)

        @pl.when(cur >= 0)
        def _():
            finalize(cur, d)

    mesh = plsc.VectorSubcoreMesh(
        core_axis_name="c", subcore_axis_name="s", num_cores=_NC, num_subcores=_NS
    )
    return pl.kernel(
        body,
        out_type=jax.ShapeDtypeStruct((n, hc), jnp.float32),
        mesh=mesh,
        scratch_types=[
            pltpu.VMEM((_ECAP,), jnp.int32),
            pltpu.VMEM((_ECAP,), jnp.int32),
            pltpu.VMEM((hc,), jnp.float32),
            pltpu.VMEM((16,), jnp.int32),
            pltpu.VMEM((2, _ECH, hc), jnp.float32),
            pltpu.VMEM((2, _ECH, hc), jnp.float32),
            pltpu.VMEM((hc,), jnp.float32),
            pltpu.SemaphoreType.DMA,
            pltpu.SemaphoreType.DMA,
            pltpu.SemaphoreType.DMA,
            pltpu.SemaphoreType.DMA,
        ],
    )


def _edge_phase_sc(xl, xr, src_s, dst_s, meta, att_flat, heads, outc):
    n = xl.shape[0]
    fn = _make_sc_edge(heads, outc, n)
    return fn(xl, xr, src_s, dst_s, meta, att_flat)


def _prep_edges(edge_index, n):
    ar = jnp.arange(n, dtype=edge_index.dtype)
    ei = jnp.concatenate([edge_index, jnp.stack([ar, ar])], axis=1)
    src, dst = ei[0], ei[1]
    order = jnp.argsort(dst)
    src_s = src[order]
    dst_s = dst[order]
    e = src_s.shape[0]
    pad = _ECAP + 16
    src_s = jnp.concatenate([src_s, jnp.zeros((pad,), src_s.dtype)])
    dst_s = jnp.concatenate([dst_s, jnp.full((pad,), n - 1, dst_s.dtype)])
    bounds = jnp.minimum(jnp.arange(_NW + 1, dtype=jnp.int32) * _NPT, n)
    offs = jnp.searchsorted(dst_s[:e], bounds).astype(jnp.int32)
    meta = jnp.zeros((_NW, 16), jnp.int32)
    meta = meta.at[:, 0].set(offs[:-1])
    meta = meta.at[:, 1].set(offs[1:])
    return src_s, dst_s, meta


def kernel(x, edge_index, W1l, b1l, W1r, b1r, att1, bias1, W2l, b2l, W2r, b2r, att2, bias2):
    n = x.shape[0]
    src_s, dst_s, meta = _prep_edges(edge_index, n)

    xl1, xr1 = _proj2(x, W1l, b1l, W1r, b1r)
    agg1 = _edge_phase_sc(xl1, xr1, src_s, dst_s, meta, att1.reshape(-1), _HEADS1, _HID)

    xl2, xr2 = _proj2(agg1, W2l, b2l, W2r, b2r, elu_in=True, bias_pre=bias1)
    agg2 = _edge_phase_sc(xl2, xr2, src_s, dst_s, meta, att2.reshape(-1), 1, _OUT)

    return agg2 + bias2


# speedup vs baseline: 6.5926x; 1.0280x over previous
import functools

import jax
import jax.numpy as jnp
from jax import lax
from jax.experimental import pallas as pl
from jax.experimental.pallas import tpu as pltpu
from jax.experimental.pallas import tpu_sc as plsc

_N = 10000
_HEADS1 = 4
_HID = 256
_OUT = 256

_NC = 2
_NS = 16
_NW = _NC * _NS
_NPT = -(-_N // _NW)
_ECAP = 8192
_ECH = 16


def _proj2(x, Wl, bl, Wr, br, elu_in=False, bias_pre=None):
    n, k = x.shape
    ko = Wl.shape[1]
    blk = 1000
    grid = n // blk

    if elu_in:
        def body(x_ref, bp_ref, wl_ref, bl_ref, wr_ref, br_ref, xl_ref, xr_ref):
            xv = x_ref[...] + bp_ref[...]
            xv = jnp.where(xv > 0, xv, jnp.exp(jnp.minimum(xv, 0.0)) - 1.0)
            xl_ref[...] = (
                jnp.dot(xv, wl_ref[...], preferred_element_type=jnp.float32)
                + bl_ref[...]
            )
            xr_ref[...] = (
                jnp.dot(xv, wr_ref[...], preferred_element_type=jnp.float32)
                + br_ref[...]
            )

        in_specs = [
            pl.BlockSpec((blk, k), lambda i: (i, 0)),
            pl.BlockSpec((1, k), lambda i: (0, 0)),
            pl.BlockSpec((k, ko), lambda i: (0, 0)),
            pl.BlockSpec((1, ko), lambda i: (0, 0)),
            pl.BlockSpec((k, ko), lambda i: (0, 0)),
            pl.BlockSpec((1, ko), lambda i: (0, 0)),
        ]
        args = (x, bias_pre.reshape(1, k), Wl, bl.reshape(1, ko), Wr, br.reshape(1, ko))
    else:
        def body(x_ref, wl_ref, bl_ref, wr_ref, br_ref, xl_ref, xr_ref):
            xv = x_ref[...]
            xl_ref[...] = (
                jnp.dot(xv, wl_ref[...], preferred_element_type=jnp.float32)
                + bl_ref[...]
            )
            xr_ref[...] = (
                jnp.dot(xv, wr_ref[...], preferred_element_type=jnp.float32)
                + br_ref[...]
            )

        in_specs = [
            pl.BlockSpec((blk, k), lambda i: (i, 0)),
            pl.BlockSpec((k, ko), lambda i: (0, 0)),
            pl.BlockSpec((1, ko), lambda i: (0, 0)),
            pl.BlockSpec((k, ko), lambda i: (0, 0)),
            pl.BlockSpec((1, ko), lambda i: (0, 0)),
        ]
        args = (x, Wl, bl.reshape(1, ko), Wr, br.reshape(1, ko))

    return pl.pallas_call(
        body,
        grid=(grid,),
        in_specs=in_specs,
        out_specs=[
            pl.BlockSpec((blk, ko), lambda i: (i, 0)),
            pl.BlockSpec((blk, ko), lambda i: (i, 0)),
        ],
        out_shape=[
            jax.ShapeDtypeStruct((n, ko), jnp.float32),
            jax.ShapeDtypeStruct((n, ko), jnp.float32),
        ],
    )(*args)


def _shufsum(v):
    for sh in (8, 4, 2, 1):
        perm = (lax.iota(jnp.int32, 16) + sh) & 15
        v = v + jnp.take(v, perm)
    return v


def _ilane(vec, k):
    sel = jnp.where(lax.iota(jnp.int32, 16) == k, vec, 0)
    return _shufsum(sel)[0]


def _make_sc_edge(heads, outc, n):
    hc = heads * outc
    cpsl = outc // 16

    def body(xl_hbm, xr_hbm, src_hbm, dst_hbm, meta_hbm, att_hbm, out_hbm,
             srcb, dstb, attb, metab, xlb, xrb, ob,
             sxl0, sxl1, sxr0, sxr1):
        sems = ((sxl0, sxr0), (sxl1, sxr1))
        wid = lax.axis_index("c") * _NS + lax.axis_index("s")

        pltpu.sync_copy(meta_hbm.at[wid], metab)
        mv = metab[...]
        e_lo = mv[0]
        e_hi = mv[1]
        base = pl.multiple_of(e_lo & ~7, 8)
        total = e_hi - base
        nch = (total + _ECH - 1) // _ECH

        pltpu.sync_copy(src_hbm.at[pl.ds(base, _ECAP)], srcb)
        pltpu.sync_copy(dst_hbm.at[pl.ds(base, _ECAP)], dstb)
        pltpu.sync_copy(att_hbm, attb)

        zeros16 = jnp.zeros((16,), jnp.float32)

        def zero_ob():
            def zb(s, c):
                ob[pl.ds(s * 16, 16)] = zeros16
                return c
            lax.fori_loop(0, hc // 16, zb, 0)

        zero_ob()

        def issue(c, b):
            @pl.when(c < nch)
            def _():
                idx = srcb[pl.ds(_ECH * c, _ECH)]
                pltpu.async_copy(xl_hbm.at[idx], xlb.at[b], sems[b][0])
                idxd = dstb[pl.ds(_ECH * c, _ECH)]
                pltpu.async_copy(xr_hbm.at[idxd], xrb.at[b], sems[b][1])

        def wait(c, b):
            @pl.when(c < nch)
            def _():
                idx = srcb[pl.ds(_ECH * c, _ECH)]
                pltpu.make_async_copy(xl_hbm.at[idx], xlb.at[b], sems[b][0]).wait()
                idxd = dstb[pl.ds(_ECH * c, _ECH)]
                pltpu.make_async_copy(xr_hbm.at[idxd], xrb.at[b], sems[b][1]).wait()

        def finalize(cur, d):
            inv_pk = 1.0 / (d + 1e-16)
            for h in range(heads):
                inv_v = jnp.take(inv_pk, jnp.full((16,), h, jnp.int32))

                def nb(s4, c):
                    for j in range(4):
                        off = h * outc + (s4 * 4 + j) * 16
                        ob[pl.ds(off, 16)] = ob[pl.ds(off, 16)] * inv_v
                    return c
                lax.fori_loop(0, cpsl // 4, nb, 0)
            pltpu.sync_copy(ob, out_hbm.at[cur])

        neg_big_v = jnp.full((16,), -1e30, jnp.float32)

        def process_chunk(c, b, carry):
            wait(c, b)
            dst16 = dstb[pl.ds(_ECH * c, _ECH)]
            ebase = base + _ECH * c
            k_lo = jnp.maximum(0, e_lo - ebase)
            k_hi = jnp.minimum(_ECH, e_hi - ebase)

            lane = lax.iota(jnp.int32, 16)

            def edge_body(k, carry):
                cur, m, d = carry
                dstv = _ilane(dst16, k)
                is_new = dstv != cur

                @pl.when(jnp.logical_and(is_new, cur >= 0))
                def _():
                    finalize(cur, d)

                m = jnp.where(is_new, neg_big_v, m)
                d = jnp.where(is_new, zeros16, d)
                cur = dstv

                e_pk = zeros16
                for h in range(heads):
                    acc = zeros16
                    for s in range(cpsl):
                        off = h * outc + s * 16
                        zl = xlb[b, k, pl.ds(off, 16)]
                        zr = xrb[b, k, pl.ds(off, 16)]
                        z = zl + zr
                        z = jnp.maximum(z, 0.2 * z)
                        acc = acc + attb[pl.ds(off, 16)] * z
                    e_pk = jnp.where(lane == h, _shufsum(acc), e_pk)

                m_new = jnp.maximum(m, e_pk)
                sc_pk = jnp.exp(m - m_new)
                w_pk = jnp.exp(e_pk - m_new)
                d_new = d * sc_pk + w_pk

                for h in range(heads):
                    scale_v = jnp.take(sc_pk, jnp.full((16,), h, jnp.int32))
                    w_v = jnp.take(w_pk, jnp.full((16,), h, jnp.int32))
                    for s in range(cpsl):
                        off = h * outc + s * 16
                        ob[pl.ds(off, 16)] = (
                            ob[pl.ds(off, 16)] * scale_v
                            + w_v * xlb[b, k, pl.ds(off, 16)]
                        )

                return cur, m_new, d_new

            carry = lax.fori_loop(k_lo, k_hi, edge_body, carry)
            issue(c + 2, b)
            return carry

        issue(0, 0)
        issue(1, 1)

        carry0 = (jnp.int32(-1), neg_big_v, zeros16)

        def pair_body(g, carry):
            carry = process_chunk(2 * g, 0, carry)
            carry = process_chunk(2 * g + 1, 1, carry)
            return carry

        cur, m, d = lax.fori_loop(0, (nch + 1) // 2, pair_body, carry0)

        @pl.when(cur >= 0)
        def _():
            finalize(cur, d)

    mesh = plsc.VectorSubcoreMesh(
        core_axis_name="c", subcore_axis_name="s", num_cores=_NC, num_subcores=_NS
    )
    return pl.kernel(
        body,
        out_type=jax.ShapeDtypeStruct((n, hc), jnp.float32),
        mesh=mesh,
        scratch_types=[
            pltpu.VMEM((_ECAP,), jnp.int32),
            pltpu.VMEM((_ECAP,), jnp.int32),
            pltpu.VMEM((hc,), jnp.float32),
            pltpu.VMEM((16,), jnp.int32),
            pltpu.VMEM((2, _ECH, hc), jnp.float32),
            pltpu.VMEM((2, _ECH, hc), jnp.float32),
            pltpu.VMEM((hc,), jnp.float32),
            pltpu.SemaphoreType.DMA,
            pltpu.SemaphoreType.DMA,
            pltpu.SemaphoreType.DMA,
            pltpu.SemaphoreType.DMA,
        ],
    )


def _edge_phase_sc(xl, xr, src_s, dst_s, meta, att_flat, heads, outc):
    n = xl.shape[0]
    fn = _make_sc_edge(heads, outc, n)
    return fn(xl, xr, src_s, dst_s, meta, att_flat)


def _prep_edges(edge_index, n):
    ar = jnp.arange(n, dtype=edge_index.dtype)
    ei = jnp.concatenate([edge_index, jnp.stack([ar, ar])], axis=1)
    src, dst = ei[0], ei[1]
    order = jnp.argsort(dst)
    src_s = src[order]
    dst_s = dst[order]
    e = src_s.shape[0]
    pad = _ECAP + 16
    src_s = jnp.concatenate([src_s, jnp.zeros((pad,), src_s.dtype)])
    dst_s = jnp.concatenate([dst_s, jnp.full((pad,), n - 1, dst_s.dtype)])
    bounds = jnp.minimum(jnp.arange(_NW + 1, dtype=jnp.int32) * _NPT, n)
    offs = jnp.searchsorted(dst_s[:e], bounds).astype(jnp.int32)
    meta = jnp.zeros((_NW, 16), jnp.int32)
    meta = meta.at[:, 0].set(offs[:-1])
    meta = meta.at[:, 1].set(offs[1:])
    return src_s, dst_s, meta


def kernel(x, edge_index, W1l, b1l, W1r, b1r, att1, bias1, W2l, b2l, W2r, b2r, att2, bias2):
    n = x.shape[0]
    src_s, dst_s, meta = _prep_edges(edge_index, n)

    xl1, xr1 = _proj2(x, W1l, b1l, W1r, b1r)
    agg1 = _edge_phase_sc(xl1, xr1, src_s, dst_s, meta, att1.reshape(-1), _HEADS1, _HID)

    xl2, xr2 = _proj2(agg1, W2l, b2l, W2r, b2r, elu_in=True, bias_pre=bias1)
    agg2 = _edge_phase_sc(xl2, xr2, src_s, dst_s, meta, att2.reshape(-1), 1, _OUT)

    return agg2 + bias2


# speedup vs baseline: 7.1229x; 1.0804x over previous
import functools

import jax
import jax.numpy as jnp
from jax import lax
from jax.experimental import pallas as pl
from jax.experimental.pallas import tpu as pltpu
from jax.experimental.pallas import tpu_sc as plsc

_N = 10000
_HEADS1 = 4
_HID = 256
_OUT = 256

_NC = 2
_NS = 16
_NW = _NC * _NS
_NPT = -(-_N // _NW)
_ECAP = 8192
_ECH = 16


def _proj2(x, Wl, bl, Wr, br, elu_in=False, bias_pre=None):
    n, k = x.shape
    ko = Wl.shape[1]
    blk = 1000
    grid = n // blk

    if elu_in:
        def body(x_ref, bp_ref, wl_ref, bl_ref, wr_ref, br_ref, xl_ref, xr_ref):
            xv = x_ref[...] + bp_ref[...]
            xv = jnp.where(xv > 0, xv, jnp.exp(jnp.minimum(xv, 0.0)) - 1.0)
            xl_ref[...] = (
                jnp.dot(xv, wl_ref[...], preferred_element_type=jnp.float32)
                + bl_ref[...]
            )
            xr_ref[...] = (
                jnp.dot(xv, wr_ref[...], preferred_element_type=jnp.float32)
                + br_ref[...]
            )

        in_specs = [
            pl.BlockSpec((blk, k), lambda i: (i, 0)),
            pl.BlockSpec((1, k), lambda i: (0, 0)),
            pl.BlockSpec((k, ko), lambda i: (0, 0)),
            pl.BlockSpec((1, ko), lambda i: (0, 0)),
            pl.BlockSpec((k, ko), lambda i: (0, 0)),
            pl.BlockSpec((1, ko), lambda i: (0, 0)),
        ]
        args = (x, bias_pre.reshape(1, k), Wl, bl.reshape(1, ko), Wr, br.reshape(1, ko))
    else:
        def body(x_ref, wl_ref, bl_ref, wr_ref, br_ref, xl_ref, xr_ref):
            xv = x_ref[...]
            xl_ref[...] = (
                jnp.dot(xv, wl_ref[...], preferred_element_type=jnp.float32)
                + bl_ref[...]
            )
            xr_ref[...] = (
                jnp.dot(xv, wr_ref[...], preferred_element_type=jnp.float32)
                + br_ref[...]
            )

        in_specs = [
            pl.BlockSpec((blk, k), lambda i: (i, 0)),
            pl.BlockSpec((k, ko), lambda i: (0, 0)),
            pl.BlockSpec((1, ko), lambda i: (0, 0)),
            pl.BlockSpec((k, ko), lambda i: (0, 0)),
            pl.BlockSpec((1, ko), lambda i: (0, 0)),
        ]
        args = (x, Wl, bl.reshape(1, ko), Wr, br.reshape(1, ko))

    return pl.pallas_call(
        body,
        grid=(grid,),
        in_specs=in_specs,
        out_specs=[
            pl.BlockSpec((blk, ko), lambda i: (i, 0)),
            pl.BlockSpec((blk, ko), lambda i: (i, 0)),
        ],
        out_shape=[
            jax.ShapeDtypeStruct((n, ko), jnp.float32),
            jax.ShapeDtypeStruct((n, ko), jnp.float32),
        ],
    )(*args)


def _shufsum(v):
    for sh in (8, 4, 2, 1):
        perm = (lax.iota(jnp.int32, 16) + sh) & 15
        v = v + jnp.take(v, perm)
    return v


def _ilane(vec, k):
    sel = jnp.where(lax.iota(jnp.int32, 16) == k, vec, 0)
    return _shufsum(sel)[0]


def _make_sc_edge(heads, outc, n):
    hc = heads * outc
    cpsl = outc // 16

    def body(xl_hbm, xr_hbm, src_hbm, dst_hbm, meta_hbm, att_hbm, out_hbm,
             srcb, dstb, attb, metab, xlb, xrb, ob,
             sxl0, sxl1, sxr0, sxr1):
        sems = ((sxl0, sxr0), (sxl1, sxr1))
        wid = lax.axis_index("c") * _NS + lax.axis_index("s")

        pltpu.sync_copy(meta_hbm.at[wid], metab)
        mv = metab[...]
        e_lo = mv[0]
        e_hi = mv[1]
        base = pl.multiple_of(e_lo & ~7, 8)
        total = e_hi - base
        nch = (total + _ECH - 1) // _ECH

        pltpu.sync_copy(src_hbm.at[pl.ds(base, _ECAP)], srcb)
        pltpu.sync_copy(dst_hbm.at[pl.ds(base, _ECAP)], dstb)
        pltpu.sync_copy(att_hbm, attb)

        zeros16 = jnp.zeros((16,), jnp.float32)

        def zero_ob():
            def zb(s, c):
                ob[pl.ds(s * 16, 16)] = zeros16
                return c
            lax.fori_loop(0, hc // 16, zb, 0)

        zero_ob()

        def issue(c, b):
            @pl.when(c < nch)
            def _():
                idx = srcb[pl.ds(_ECH * c, _ECH)]
                pltpu.async_copy(xl_hbm.at[idx], xlb.at[b], sems[b][0])
                idxd = dstb[pl.ds(_ECH * c, _ECH)]
                pltpu.async_copy(xr_hbm.at[idxd], xrb.at[b], sems[b][1])

        def wait(c, b):
            @pl.when(c < nch)
            def _():
                idx = srcb[pl.ds(_ECH * c, _ECH)]
                pltpu.make_async_copy(xl_hbm.at[idx], xlb.at[b], sems[b][0]).wait()
                idxd = dstb[pl.ds(_ECH * c, _ECH)]
                pltpu.make_async_copy(xr_hbm.at[idxd], xrb.at[b], sems[b][1]).wait()

        def finalize(cur, d):
            inv_pk = 1.0 / (d + 1e-16)
            for h in range(heads):
                inv_v = jnp.take(inv_pk, jnp.full((16,), h, jnp.int32))

                def nb(s4, c):
                    for j in range(4):
                        off = h * outc + (s4 * 4 + j) * 16
                        ob[pl.ds(off, 16)] = ob[pl.ds(off, 16)] * inv_v
                    return c
                lax.fori_loop(0, cpsl // 4, nb, 0)
            pltpu.sync_copy(ob, out_hbm.at[cur])
            zero_ob()


        def process_chunk(c, b, carry):
            wait(c, b)
            dst16 = dstb[pl.ds(_ECH * c, _ECH)]
            ebase = base + _ECH * c
            k_lo = jnp.maximum(0, e_lo - ebase)
            k_hi = jnp.minimum(_ECH, e_hi - ebase)

            lane = lax.iota(jnp.int32, 16)

            def edge_body(k, carry):
                cur, d = carry
                dstv = _ilane(dst16, k)
                is_new = dstv != cur

                @pl.when(jnp.logical_and(is_new, cur >= 0))
                def _():
                    finalize(cur, d)

                d = jnp.where(is_new, zeros16, d)
                cur = dstv

                e_pk = zeros16
                for h in range(heads):
                    acc = zeros16
                    for s in range(cpsl):
                        off = h * outc + s * 16
                        zl = xlb[b, k, pl.ds(off, 16)]
                        zr = xrb[b, k, pl.ds(off, 16)]
                        z = zl + zr
                        z = jnp.maximum(z, 0.2 * z)
                        acc = acc + attb[pl.ds(off, 16)] * z
                    e_pk = jnp.where(lane == h, _shufsum(acc), e_pk)

                w_pk = jnp.exp(e_pk)
                d_new = d + w_pk

                for h in range(heads):
                    w_v = jnp.take(w_pk, jnp.full((16,), h, jnp.int32))
                    for s in range(cpsl):
                        off = h * outc + s * 16
                        plsc.addupdate(
                            ob.at[pl.ds(off, 16)],
                            w_v * xlb[b, k, pl.ds(off, 16)],
                        )

                return cur, d_new

            carry = lax.fori_loop(k_lo, k_hi, edge_body, carry)
            issue(c + 2, b)
            return carry

        issue(0, 0)
        issue(1, 1)

        carry0 = (jnp.int32(-1), zeros16)

        def pair_body(g, carry):
            carry = process_chunk(2 * g, 0, carry)
            carry = process_chunk(2 * g + 1, 1, carry)
            return carry

        cur, d = lax.fori_loop(0, (nch + 1) // 2, pair_body, carry0)

        @pl.when(cur >= 0)
        def _():
            finalize(cur, d)

    mesh = plsc.VectorSubcoreMesh(
        core_axis_name="c", subcore_axis_name="s", num_cores=_NC, num_subcores=_NS
    )
    return pl.kernel(
        body,
        out_type=jax.ShapeDtypeStruct((n, hc), jnp.float32),
        mesh=mesh,
        scratch_types=[
            pltpu.VMEM((_ECAP,), jnp.int32),
            pltpu.VMEM((_ECAP,), jnp.int32),
            pltpu.VMEM((hc,), jnp.float32),
            pltpu.VMEM((16,), jnp.int32),
            pltpu.VMEM((2, _ECH, hc), jnp.float32),
            pltpu.VMEM((2, _ECH, hc), jnp.float32),
            pltpu.VMEM((hc,), jnp.float32),
            pltpu.SemaphoreType.DMA,
            pltpu.SemaphoreType.DMA,
            pltpu.SemaphoreType.DMA,
            pltpu.SemaphoreType.DMA,
        ],
    )


def _edge_phase_sc(xl, xr, src_s, dst_s, meta, att_flat, heads, outc):
    n = xl.shape[0]
    fn = _make_sc_edge(heads, outc, n)
    return fn(xl, xr, src_s, dst_s, meta, att_flat)


def _prep_edges(edge_index, n):
    ar = jnp.arange(n, dtype=edge_index.dtype)
    ei = jnp.concatenate([edge_index, jnp.stack([ar, ar])], axis=1)
    src, dst = ei[0], ei[1]
    order = jnp.argsort(dst)
    src_s = src[order]
    dst_s = dst[order]
    e = src_s.shape[0]
    pad = _ECAP + 16
    src_s = jnp.concatenate([src_s, jnp.zeros((pad,), src_s.dtype)])
    dst_s = jnp.concatenate([dst_s, jnp.full((pad,), n - 1, dst_s.dtype)])
    bounds = jnp.minimum(jnp.arange(_NW + 1, dtype=jnp.int32) * _NPT, n)
    offs = jnp.searchsorted(dst_s[:e], bounds).astype(jnp.int32)
    meta = jnp.zeros((_NW, 16), jnp.int32)
    meta = meta.at[:, 0].set(offs[:-1])
    meta = meta.at[:, 1].set(offs[1:])
    return src_s, dst_s, meta


def kernel(x, edge_index, W1l, b1l, W1r, b1r, att1, bias1, W2l, b2l, W2r, b2r, att2, bias2):
    n = x.shape[0]
    src_s, dst_s, meta = _prep_edges(edge_index, n)

    xl1, xr1 = _proj2(x, W1l, b1l, W1r, b1r)
    agg1 = _edge_phase_sc(xl1, xr1, src_s, dst_s, meta, att1.reshape(-1), _HEADS1, _HID)

    xl2, xr2 = _proj2(agg1, W2l, b2l, W2r, b2r, elu_in=True, bias_pre=bias1)
    agg2 = _edge_phase_sc(xl2, xr2, src_s, dst_s, meta, att2.reshape(-1), 1, _OUT)

    return agg2 + bias2
